# Initial kernel scaffold; baseline (speedup 1.0000x reference)
import jax
import jax.numpy as jnp
from jax.experimental import pallas as pl


def kernel(x, h, edge_index, W1, b1, W2, b2):
    raise NotImplementedError("write your pallas kernel here")


# speedup vs baseline: 7.2878x; 7.2878x over previous
import functools

import jax
import jax.numpy as jnp
from jax import lax
from jax.experimental import pallas as pl
from jax.experimental.pallas import tpu as pltpu
from jax.experimental.pallas import tpu_sc as plsc

N = 10000
E = 320000
D = 128
DH = 64

K = 128
CC = 160
EPW = CC * K
E_PAD = 16 * EPW
ACC_ROWS = 10112
ZBLOCKS = ACC_ROWS // 128
ORPT = 624


def _sc_segsum_body(u_lo_hbm, u_hi_hbm, src_hbm, dst_hbm, out_hbm,
                    src_v, dst_v, buf0, buf1, zero_v, acc_sh, sem0, sem1):
    c = lax.axis_index("c")
    s = lax.axis_index("s")

    pltpu.sync_copy(src_hbm.at[s], src_v)
    pltpu.sync_copy(dst_hbm.at[s], dst_v)

    def _zb(i, carry):
        zero_v[i // 4, pl.ds((i % 4) * 16, 16)] = jnp.zeros((16,), jnp.float32)
        return carry
    lax.fori_loop(0, 128 * 4, _zb, 0)

    for kblk in range(5):
        b = s + 16 * kblk
        if kblk < 4:
            pltpu.sync_copy(zero_v, acc_sh.at[pl.ds(b * 128, 128)])
        else:
            @pl.when(s < ZBLOCKS - 64)
            def _():
                pltpu.sync_copy(zero_v, acc_sh.at[pl.ds(b * 128, 128)])
    plsc.subcore_barrier()

    def _run(u_hbm):
        def _chunk_pair(j2, carry):
            j = j2 * 2
            cp0 = pltpu.async_copy(u_hbm.at[src_v.at[j]], buf0, sem0)
            cp1 = pltpu.async_copy(u_hbm.at[src_v.at[j + 1]], buf1, sem1)
            cp0.wait()
            pltpu.sync_copy(buf0, acc_sh.at[dst_v.at[j]], add=True)
            cp1.wait()
            pltpu.sync_copy(buf1, acc_sh.at[dst_v.at[j + 1]], add=True)
            return carry
        lax.fori_loop(0, CC // 2, _chunk_pair, 0)

    @pl.when(c == 0)
    def _():
        _run(u_lo_hbm)

    @pl.when(c == 1)
    def _():
        _run(u_hi_hbm)

    plsc.subcore_barrier()
    pltpu.sync_copy(acc_sh.at[pl.ds(s * ORPT, ORPT)],
                    out_hbm.at[c, pl.ds(s * ORPT, ORPT)])

    @pl.when(s == 15)
    def _():
        tail = 16 * ORPT
        pltpu.sync_copy(acc_sh.at[pl.ds(tail, N - tail)],
                        out_hbm.at[c, pl.ds(tail, N - tail)])


@functools.cache
def _sc_segsum():
    return pl.kernel(
        _sc_segsum_body,
        out_type=jax.ShapeDtypeStruct((2, N, DH), jnp.float32),
        mesh=plsc.VectorSubcoreMesh(core_axis_name="c", subcore_axis_name="s"),
        compiler_params=pltpu.CompilerParams(use_tc_tiling_on_sc=False),
        scratch_types=[
            pltpu.VMEM((CC, K), jnp.int32),
            pltpu.VMEM((CC, K), jnp.int32),
            pltpu.VMEM((K, DH), jnp.float32),
            pltpu.VMEM((K, DH), jnp.float32),
            pltpu.VMEM((128, DH), jnp.float32),
            pltpu.VMEM_SHARED((ACC_ROWS, DH), jnp.float32),
            pltpu.SemaphoreType.DMA,
            pltpu.SemaphoreType.DMA,
        ],
    )


def _tc1_body(x_ref, h_ref, wa_ref, wb_ref, ulo_ref, uhi_ref):
    u = (jnp.dot(x_ref[...], wa_ref[...], preferred_element_type=jnp.float32)
         + jnp.dot(h_ref[...], wb_ref[...], preferred_element_type=jnp.float32))
    ulo_ref[...] = u[:, :DH]
    uhi_ref[...] = u[:, DH:]


def _tc2_body(x_ref, s0_ref, s1_ref, wx_ref, b1_ref, w2_ref, b2_ref, o_ref):
    m1 = jnp.concatenate([s0_ref[...], s1_ref[...]], axis=1)
    z = (jnp.dot(x_ref[...], wx_ref[...], preferred_element_type=jnp.float32)
         + m1 + b1_ref[...])
    hid = jnp.maximum(z, 0.0)
    o_ref[...] = (jnp.dot(hid, w2_ref[...], preferred_element_type=jnp.float32)
                  + b2_ref[...])


_ROWS_BLK = 1000


def kernel(x, h, edge_index, W1, b1, W2, b2):
    wx_t = W1[:, :D].T
    wa_t = W1[:, D:2 * D].T
    wb_t = W1[:, 2 * D:].T
    w2_t = W2.T

    grid = (N // _ROWS_BLK,)
    row_spec = pl.BlockSpec((_ROWS_BLK, D), lambda i: (i, 0))
    half_spec = pl.BlockSpec((_ROWS_BLK, DH), lambda i: (i, 0))
    full_spec = pl.BlockSpec((D, D), lambda i: (0, 0))
    bias_spec = pl.BlockSpec((1, D), lambda i: (0, 0))

    u_lo, u_hi = pl.pallas_call(
        _tc1_body,
        grid=grid,
        in_specs=[row_spec, row_spec, full_spec, full_spec],
        out_specs=[half_spec, half_spec],
        out_shape=[jax.ShapeDtypeStruct((N, DH), jnp.float32),
                   jax.ShapeDtypeStruct((N, DH), jnp.float32)],
    )(x, h, wa_t, wb_t)

    pad = E_PAD - E
    src3 = jnp.concatenate(
        [edge_index[0], jnp.zeros((pad,), jnp.int32)]).reshape(16, CC, K)
    dst3 = jnp.concatenate(
        [edge_index[1], jnp.full((pad,), N, jnp.int32)]).reshape(16, CC, K)

    parts = _sc_segsum()(u_lo, u_hi, src3, dst3)

    out = pl.pallas_call(
        _tc2_body,
        grid=grid,
        in_specs=[row_spec, half_spec, half_spec, full_spec, bias_spec,
                  full_spec, bias_spec],
        out_specs=row_spec,
        out_shape=jax.ShapeDtypeStruct((N, D), jnp.float32),
    )(x, parts[0], parts[1], wx_t, b1.reshape(1, D), w2_t, b2.reshape(1, D))
    return out


# speedup vs baseline: 8.3990x; 1.1525x over previous
import functools

import jax
import jax.numpy as jnp
from jax import lax
from jax.experimental import pallas as pl
from jax.experimental.pallas import tpu as pltpu
from jax.experimental.pallas import tpu_sc as plsc

N = 10000
E = 320000
D = 128
DH = 64

K = 128
CC = 160
EPW = CC * K
E_PAD = 16 * EPW
ACC_ROWS = 10112
ZBLOCKS = ACC_ROWS // 128
ORPT = 624


def _sc_segsum_body(u_lo_hbm, u_hi_hbm, src_hbm, dst_hbm, out_hbm,
                    src_v, dst_v, buf0, buf1, buf2, buf3, zero_v, acc_sh,
                    gsem0, gsem1, gsem2, gsem3, ssem0, ssem1, ssem2, ssem3):
    c = lax.axis_index("c")
    s = lax.axis_index("s")

    pltpu.sync_copy(src_hbm.at[s], src_v)
    pltpu.sync_copy(dst_hbm.at[s], dst_v)

    def _zb(i, carry):
        zero_v[i // 4, pl.ds((i % 4) * 16, 16)] = jnp.zeros((16,), jnp.float32)
        return carry
    lax.fori_loop(0, 128 * 4, _zb, 0)

    for kblk in range(5):
        b = s + 16 * kblk
        if kblk < 4:
            pltpu.sync_copy(zero_v, acc_sh.at[pl.ds(b * 128, 128)])
        else:
            @pl.when(s < ZBLOCKS - 64)
            def _():
                pltpu.sync_copy(zero_v, acc_sh.at[pl.ds(b * 128, 128)])
    plsc.subcore_barrier()

    bufs = (buf0, buf1, buf2, buf3)
    gsems = (gsem0, gsem1, gsem2, gsem3)
    ssems = (ssem0, ssem1, ssem2, ssem3)
    NB = 4
    NITER = CC // NB

    def _run(u_hbm):
        for b in range(NB):
            pltpu.async_copy(u_hbm.at[src_v.at[b]], bufs[b], gsems[b])

        def _iter(i, carry):
            for b in range(NB):
                pltpu.make_async_copy(
                    u_hbm.at[src_v.at[0]], bufs[b], gsems[b]).wait()
                pltpu.async_copy(bufs[b], acc_sh.at[dst_v.at[i * NB + b]],
                                 ssems[b], add=True)
            for b in range(NB):
                pltpu.make_async_copy(
                    bufs[b], acc_sh.at[dst_v.at[0]], ssems[b]).wait()

                @pl.when(i < NITER - 1)
                def _():
                    pltpu.async_copy(u_hbm.at[src_v.at[(i + 1) * NB + b]],
                                     bufs[b], gsems[b])
            return carry
        lax.fori_loop(0, NITER, _iter, 0)

    @pl.when(c == 0)
    def _():
        _run(u_lo_hbm)

    @pl.when(c == 1)
    def _():
        _run(u_hi_hbm)

    plsc.subcore_barrier()
    pltpu.sync_copy(acc_sh.at[pl.ds(s * ORPT, ORPT)],
                    out_hbm.at[c, pl.ds(s * ORPT, ORPT)])

    @pl.when(s == 15)
    def _():
        tail = 16 * ORPT
        pltpu.sync_copy(acc_sh.at[pl.ds(tail, N - tail)],
                        out_hbm.at[c, pl.ds(tail, N - tail)])


@functools.cache
def _sc_segsum():
    return pl.kernel(
        _sc_segsum_body,
        out_type=jax.ShapeDtypeStruct((2, N, DH), jnp.float32),
        mesh=plsc.VectorSubcoreMesh(core_axis_name="c", subcore_axis_name="s"),
        compiler_params=pltpu.CompilerParams(use_tc_tiling_on_sc=False),
        scratch_types=[
            pltpu.VMEM((CC, K), jnp.int32),
            pltpu.VMEM((CC, K), jnp.int32),
            pltpu.VMEM((K, DH), jnp.float32),
            pltpu.VMEM((K, DH), jnp.float32),
            pltpu.VMEM((K, DH), jnp.float32),
            pltpu.VMEM((K, DH), jnp.float32),
            pltpu.VMEM((128, DH), jnp.float32),
            pltpu.VMEM_SHARED((ACC_ROWS, DH), jnp.float32),
            pltpu.SemaphoreType.DMA,
            pltpu.SemaphoreType.DMA,
            pltpu.SemaphoreType.DMA,
            pltpu.SemaphoreType.DMA,
            pltpu.SemaphoreType.DMA,
            pltpu.SemaphoreType.DMA,
            pltpu.SemaphoreType.DMA,
            pltpu.SemaphoreType.DMA,
        ],
    )


def _tc1_body(x_ref, h_ref, wa_ref, wb_ref, ulo_ref, uhi_ref):
    u = (jnp.dot(x_ref[...], wa_ref[...], preferred_element_type=jnp.float32)
         + jnp.dot(h_ref[...], wb_ref[...], preferred_element_type=jnp.float32))
    ulo_ref[...] = u[:, :DH]
    uhi_ref[...] = u[:, DH:]


def _tc2_body(x_ref, s0_ref, s1_ref, wx_ref, b1_ref, w2_ref, b2_ref, o_ref):
    m1 = jnp.concatenate([s0_ref[...], s1_ref[...]], axis=1)
    z = (jnp.dot(x_ref[...], wx_ref[...], preferred_element_type=jnp.float32)
         + m1 + b1_ref[...])
    hid = jnp.maximum(z, 0.0)
    o_ref[...] = (jnp.dot(hid, w2_ref[...], preferred_element_type=jnp.float32)
                  + b2_ref[...])


_ROWS_BLK = 1000


def kernel(x, h, edge_index, W1, b1, W2, b2):
    wx_t = W1[:, :D].T
    wa_t = W1[:, D:2 * D].T
    wb_t = W1[:, 2 * D:].T
    w2_t = W2.T

    grid = (N // _ROWS_BLK,)
    row_spec = pl.BlockSpec((_ROWS_BLK, D), lambda i: (i, 0))
    half_spec = pl.BlockSpec((_ROWS_BLK, DH), lambda i: (i, 0))
    full_spec = pl.BlockSpec((D, D), lambda i: (0, 0))
    bias_spec = pl.BlockSpec((1, D), lambda i: (0, 0))

    u_lo, u_hi = pl.pallas_call(
        _tc1_body,
        grid=grid,
        in_specs=[row_spec, row_spec, full_spec, full_spec],
        out_specs=[half_spec, half_spec],
        out_shape=[jax.ShapeDtypeStruct((N, DH), jnp.float32),
                   jax.ShapeDtypeStruct((N, DH), jnp.float32)],
    )(x, h, wa_t, wb_t)

    pad = E_PAD - E
    src3 = jnp.concatenate(
        [edge_index[0], jnp.zeros((pad,), jnp.int32)]).reshape(16, CC, K)
    dst3 = jnp.concatenate(
        [edge_index[1], jnp.full((pad,), N, jnp.int32)]).reshape(16, CC, K)

    parts = _sc_segsum()(u_lo, u_hi, src3, dst3)

    out = pl.pallas_call(
        _tc2_body,
        grid=grid,
        in_specs=[row_spec, half_spec, half_spec, full_spec, bias_spec,
                  full_spec, bias_spec],
        out_specs=row_spec,
        out_shape=jax.ShapeDtypeStruct((N, D), jnp.float32),
    )(x, parts[0], parts[1], wx_t, b1.reshape(1, D), w2_t, b2.reshape(1, D))
    return out


# speedup vs baseline: 8.4284x; 1.0035x over previous
import functools

import jax
import jax.numpy as jnp
from jax import lax
from jax.experimental import pallas as pl
from jax.experimental.pallas import tpu as pltpu
from jax.experimental.pallas import tpu_sc as plsc

N = 10000
E = 320000
D = 128
DH = 64

K = 128
CC = 160
EPW = CC * K
E_PAD = 16 * EPW
ACC_ROWS = 10112
ZBLOCKS = ACC_ROWS // 128
ORPT = 624


NB = 5


def _sc_segsum_body(u_lo_hbm, u_hi_hbm, src_hbm, dst_hbm, out_hbm,
                    src_v, dst_v, *rest):
    bufs = rest[:NB]
    acc_sh = rest[NB]
    gsems = rest[NB + 1:2 * NB + 1]
    ssems = rest[2 * NB + 1:3 * NB + 1]
    zero_v = bufs[0]
    c = lax.axis_index("c")
    s = lax.axis_index("s")

    pltpu.sync_copy(src_hbm.at[s], src_v)
    pltpu.sync_copy(dst_hbm.at[s], dst_v)

    def _zb(i, carry):
        zero_v[i // 4, pl.ds((i % 4) * 16, 16)] = jnp.zeros((16,), jnp.float32)
        return carry
    lax.fori_loop(0, 128 * 4, _zb, 0)

    for kblk in range(5):
        b = s + 16 * kblk
        if kblk < 4:
            pltpu.sync_copy(zero_v, acc_sh.at[pl.ds(b * 128, 128)])
        else:
            @pl.when(s < ZBLOCKS - 64)
            def _():
                pltpu.sync_copy(zero_v, acc_sh.at[pl.ds(b * 128, 128)])
    plsc.subcore_barrier()

    NITER = CC // NB

    def _run(u_hbm):
        for b in range(NB):
            pltpu.async_copy(u_hbm.at[src_v.at[b]], bufs[b], gsems[b])

        def _iter(i, carry):
            for b in range(NB):
                pltpu.make_async_copy(
                    u_hbm.at[src_v.at[0]], bufs[b], gsems[b]).wait()
                pltpu.async_copy(bufs[b], acc_sh.at[dst_v.at[i * NB + b]],
                                 ssems[b], add=True)
            for b in range(NB):
                pltpu.make_async_copy(
                    bufs[b], acc_sh.at[dst_v.at[0]], ssems[b]).wait()

                @pl.when(i < NITER - 1)
                def _():
                    pltpu.async_copy(u_hbm.at[src_v.at[(i + 1) * NB + b]],
                                     bufs[b], gsems[b])
            return carry
        lax.fori_loop(0, NITER, _iter, 0)

    @pl.when(c == 0)
    def _():
        _run(u_lo_hbm)

    @pl.when(c == 1)
    def _():
        _run(u_hi_hbm)

    plsc.subcore_barrier()
    pltpu.sync_copy(acc_sh.at[pl.ds(s * ORPT, ORPT)],
                    out_hbm.at[c, pl.ds(s * ORPT, ORPT)])

    @pl.when(s == 15)
    def _():
        tail = 16 * ORPT
        pltpu.sync_copy(acc_sh.at[pl.ds(tail, N - tail)],
                        out_hbm.at[c, pl.ds(tail, N - tail)])


@functools.cache
def _sc_segsum():
    return pl.kernel(
        _sc_segsum_body,
        out_type=jax.ShapeDtypeStruct((2, N, DH), jnp.float32),
        mesh=plsc.VectorSubcoreMesh(core_axis_name="c", subcore_axis_name="s"),
        compiler_params=pltpu.CompilerParams(use_tc_tiling_on_sc=False),
        scratch_types=[
            pltpu.VMEM((CC, K), jnp.int32),
            pltpu.VMEM((CC, K), jnp.int32),
            *[pltpu.VMEM((K, DH), jnp.float32) for _ in range(NB)],
            pltpu.VMEM_SHARED((ACC_ROWS, DH), jnp.float32),
            *[pltpu.SemaphoreType.DMA for _ in range(2 * NB)],
        ],
    )


def _tc1_body(x_ref, h_ref, wa_ref, wb_ref, ulo_ref, uhi_ref):
    u = (jnp.dot(x_ref[...], wa_ref[...], preferred_element_type=jnp.float32)
         + jnp.dot(h_ref[...], wb_ref[...], preferred_element_type=jnp.float32))
    ulo_ref[...] = u[:, :DH]
    uhi_ref[...] = u[:, DH:]


def _tc2_body(x_ref, s0_ref, s1_ref, wx_ref, b1_ref, w2_ref, b2_ref, o_ref):
    m1 = jnp.concatenate([s0_ref[...], s1_ref[...]], axis=1)
    z = (jnp.dot(x_ref[...], wx_ref[...], preferred_element_type=jnp.float32)
         + m1 + b1_ref[...])
    hid = jnp.maximum(z, 0.0)
    o_ref[...] = (jnp.dot(hid, w2_ref[...], preferred_element_type=jnp.float32)
                  + b2_ref[...])


_ROWS_BLK = 1000


def kernel(x, h, edge_index, W1, b1, W2, b2):
    wx_t = W1[:, :D].T
    wa_t = W1[:, D:2 * D].T
    wb_t = W1[:, 2 * D:].T
    w2_t = W2.T

    grid = (N // _ROWS_BLK,)
    row_spec = pl.BlockSpec((_ROWS_BLK, D), lambda i: (i, 0))
    half_spec = pl.BlockSpec((_ROWS_BLK, DH), lambda i: (i, 0))
    full_spec = pl.BlockSpec((D, D), lambda i: (0, 0))
    bias_spec = pl.BlockSpec((1, D), lambda i: (0, 0))

    u_lo, u_hi = pl.pallas_call(
        _tc1_body,
        grid=grid,
        in_specs=[row_spec, row_spec, full_spec, full_spec],
        out_specs=[half_spec, half_spec],
        out_shape=[jax.ShapeDtypeStruct((N, DH), jnp.float32),
                   jax.ShapeDtypeStruct((N, DH), jnp.float32)],
    )(x, h, wa_t, wb_t)

    pad = E_PAD - E
    src3 = jnp.concatenate(
        [edge_index[0], jnp.zeros((pad,), jnp.int32)]).reshape(16, CC, K)
    dst3 = jnp.concatenate(
        [edge_index[1], jnp.full((pad,), N, jnp.int32)]).reshape(16, CC, K)

    parts = _sc_segsum()(u_lo, u_hi, src3, dst3)

    out = pl.pallas_call(
        _tc2_body,
        grid=grid,
        in_specs=[row_spec, half_spec, half_spec, full_spec, bias_spec,
                  full_spec, bias_spec],
        out_specs=row_spec,
        out_shape=jax.ShapeDtypeStruct((N, D), jnp.float32),
    )(x, parts[0], parts[1], wx_t, b1.reshape(1, D), w2_t, b2.reshape(1, D))
    return out
